# Initial kernel scaffold; baseline (speedup 1.0000x reference)
import jax
import jax.numpy as jnp
from jax.experimental import pallas as pl


def kernel(coords, h, flow_dir, params):
    raise NotImplementedError("write your pallas kernel here")


# speedup vs baseline: 1.0667x; 1.0667x over previous
import jax
import jax.numpy as jnp
from jax.experimental import pallas as pl
from jax.experimental.pallas import tpu as pltpu

HID = 128
MSG = 256
RADIUS = 0.1
MAXK = 32
N = 10000
NP = 10240
BN = 64
GW = 144


def _gelu(x):
    return 0.5 * x * (1.0 + jax.lax.erf(x * 0.7071067811865476))


def _fused_body(feat_ref, g_ref, gf_ref,
                wei_ref, wej_ref, we1_ref, we2_ref,
                wg0_ref, g1w_ref,
                wu0a_ref, wu0b_ref, wu0c_ref, wu1_ref, wu2_ref,
                gl0_ref, gl1_ref, gl2_ref, glb_ref,
                s_ref, bias_ref, out_ref):
    be = BN * MAXK
    dot = lambda a, b: jax.lax.dot_general(
        a, b, (((1,), (0,)), ((), ())),
        precision=jax.lax.Precision.HIGHEST,
        preferred_element_type=jnp.float32)

    fx = feat_ref[...]
    feat = fx[:, :HID]
    xdu = fx[:, HID:HID + 1]
    g = g_ref[...]
    fj = g[:, :HID]
    d2c = g[:, HID:HID + 1]
    xju = g[:, HID + 1:HID + 2]
    vcol = g[:, HID + 2:HID + 3]

    bias = bias_ref[...]
    s = s_ref[...]
    b0 = bias[0:1]
    b1 = bias[1:2]
    b2 = bias[2:3]
    g0b = bias[3:4]
    u0b = bias[4:5]
    u1b = bias[5:6]
    u2b = bias[6:7, :HID]
    g1b = bias[7, 0]
    sres = bias[7, 1]
    w_d2 = s[0:1]
    w_a = s[1:2]
    w_b = s[2:3]

    nodepart = dot(feat, wei_ref[...]) + xdu * w_a
    npart_e = jnp.broadcast_to(
        nodepart[:, None, :], (BN, MAXK, MSG)).reshape(be, MSG)
    pre0 = npart_e + dot(fj, wej_ref[...]) + d2c * w_d2 + xju * w_b + b0
    h1 = _gelu(pre0)
    h2 = _gelu(dot(h1, we1_ref[...]) + b1)
    eh = dot(h2, we2_ref[...]) + b2

    gg = _gelu(dot(eh, wg0_ref[...]) + g0b)
    logit = jnp.sum(gg * g1w_ref[...], axis=1, keepdims=True) + g1b
    gate = jax.nn.sigmoid(logit)
    msg = eh * (gate * vcol)

    msum = jnp.sum(msg.reshape(BN, MAXK, MSG), axis=1)
    deg = jnp.sum(vcol.reshape(BN, MAXK, 1), axis=1)
    agg = msum / jnp.maximum(deg, 1.0)

    gf = gf_ref[...]
    c1 = _gelu(dot(gf, gl0_ref[...]) + glb_ref[0:1])
    c2 = _gelu(dot(c1, gl1_ref[...]) + glb_ref[1:2])
    gctx = dot(c2, gl2_ref[...]) + glb_ref[2:3]
    gterm = dot(gctx[0:1], wu0c_ref[...])

    upre = dot(feat, wu0a_ref[...]) + dot(agg, wu0b_ref[...]) + gterm + u0b
    v1 = _gelu(upre)
    v2 = _gelu(dot(v1, wu1_ref[...]) + u1b)
    dh = dot(v2, wu2_ref[...]) + u2b
    out_ref[...] = dh * sres


def _run_fused(featp, gp, gf, wei, wej, we1, we2, wg0, g1w,
               wu0a, wu0b, wu0c, wu1, wu2, gl0, gl1, gl2, glb, s, bias):
    nblk = NP // BN
    be = BN * MAXK
    full = lambda shp: pl.BlockSpec(shp, lambda i: tuple(0 for _ in shp))
    grid_spec = pl.GridSpec(
        grid=(nblk,),
        in_specs=[
            pl.BlockSpec((BN, 2 * HID), lambda i: (i, 0)),
            pl.BlockSpec((be, GW), lambda i: (i, 0)),
            full((8, HID)),
            full((HID, MSG)), full((HID, MSG)),
            full((MSG, MSG)), full((MSG, MSG)),
            full((MSG, MSG)), full((1, MSG)),
            full((HID, MSG)), full((MSG, MSG)), full((HID, MSG)),
            full((MSG, MSG)), full((MSG, HID)),
            full((HID, HID)), full((HID, HID)), full((HID, HID)),
            full((8, HID)),
            full((8, MSG)), full((8, MSG)),
        ],
        out_specs=pl.BlockSpec((BN, HID), lambda i: (i, 0)),
    )
    return pl.pallas_call(
        _fused_body,
        grid_spec=grid_spec,
        out_shape=jax.ShapeDtypeStruct((NP, HID), jnp.float32),
    )(featp, gp, gf, wei, wej, we1, we2, wg0, g1w,
      wu0a, wu0b, wu0c, wu1, wu2, gl0, gl1, gl2, glb, s, bias)


def kernel(coords, h, flow_dir, params):
    p = params
    f32 = jnp.float32
    x = coords[0]
    h0 = h[0]

    sq = jnp.sum(x * x, axis=-1)
    d2 = sq[:, None] + sq[None, :] - 2.0 * (x @ x.T)
    d2 = jnp.maximum(d2, 0.0)
    invalid = (d2 > RADIUS * RADIUS) | jnp.eye(N, dtype=bool)
    neg = jnp.where(invalid, -jnp.inf, -d2)
    vals, idx = jax.lax.top_k(neg, MAXK)
    valid = (vals > -jnp.inf).reshape(-1)
    col = idx.reshape(-1)

    mean = jnp.mean(h0, axis=-1, keepdims=True)
    var = jnp.var(h0, axis=-1, keepdims=True)
    feat = (h0 - mean) / jnp.sqrt(var + 1e-05) * p['ln_g'] + p['ln_b']
    u = flow_dir[0]
    u = u / (jnp.linalg.norm(u) + 1e-08)
    xdu = x @ u

    fj = feat[col]
    xjuv = xdu[col]
    dist2 = jnp.where(valid, jnp.maximum(-vals.reshape(-1), 0.0), 0.0)
    vf = valid.astype(f32)
    g = jnp.concatenate(
        [fj, dist2[:, None], xjuv[:, None], vf[:, None],
         jnp.zeros((N * MAXK, GW - HID - 3), f32)], axis=1)

    gp = jnp.pad(g, ((0, (NP - N) * MAXK), (0, 0)))
    fx = jnp.concatenate(
        [feat, xdu[:, None], jnp.zeros((N, HID - 1), f32)], axis=1)
    featp = jnp.pad(fx, ((0, NP - N), (0, 0)))
    gf = jnp.broadcast_to(jnp.mean(feat, axis=0, keepdims=True), (8, HID))

    e0w = p['e0_W']
    wei = e0w[0:HID] + e0w[2 * HID:3 * HID]
    wej = e0w[HID:2 * HID] - e0w[2 * HID:3 * HID]
    w_d2 = e0w[3 * HID + 0]
    w_a = e0w[3 * HID + 1] + e0w[3 * HID + 3]
    w_b = e0w[3 * HID + 2] - e0w[3 * HID + 3]
    s = jnp.zeros((8, MSG), f32).at[0].set(w_d2).at[1].set(w_a).at[2].set(w_b)
    bias = (jnp.zeros((8, MSG), f32)
            .at[0].set(p['e0_b']).at[1].set(p['e1_b']).at[2].set(p['e2_b'])
            .at[3].set(p['g0_b']).at[4].set(p['u0_b']).at[5].set(p['u1_b'])
            .at[6, :HID].set(p['u2_b'])
            .at[7, 0].set(p['g1_b'][0])
            .at[7, 1].set(jnp.tanh(p['res_scale'])))
    glb = (jnp.zeros((8, HID), f32)
           .at[0].set(p['gl0_b']).at[1].set(p['gl1_b']).at[2].set(p['gl2_b']))
    g1w = p['g1_W'].reshape(1, MSG)
    u0w = p['u0_W']
    wu0a = u0w[0:HID]
    wu0b = u0w[HID:HID + MSG]
    wu0c = u0w[HID + MSG:]

    out = _run_fused(featp, gp, gf, wei, wej, p['e1_W'], p['e2_W'],
                     p['g0_W'], g1w, wu0a, wu0b, wu0c, p['u1_W'], p['u2_W'],
                     p['gl0_W'], p['gl1_W'], p['gl2_W'], glb, s, bias)
    return out[:N][None]


# speedup vs baseline: 3.8422x; 3.6021x over previous
import jax
import jax.numpy as jnp
import numpy as np
from jax.experimental import pallas as pl
from jax.experimental.pallas import tpu as pltpu

HID = 128
MSG = 256
RADIUS = 0.1
MAXK = 32
N = 10000
NP = 10240
BN = 64
GW = 144
NC = 10016
KEYHI = np.int32(-16384)
KEYBIG = np.int32(0x7F7FFFFF)


def _bitonic_sort_net(n):
    net = []
    k = 2
    while k <= n:
        j = k // 2
        while j >= 1:
            for i in range(n):
                l = i ^ j
                if l > i:
                    net.append((i, l, (i & k) == 0))
            j //= 2
        k *= 2
    return net


def _bitonic_merge_net(n):
    net = []
    j = n // 2
    while j >= 1:
        for i in range(n):
            l = i ^ j
            if l > i:
                net.append((i, l, True))
        j //= 2
    return net


_SORT32 = _bitonic_sort_net(MAXK)
_MERGE32 = _bitonic_merge_net(MAXK)


def _apply_net(arr, net):
    for i, l, asc in net:
        a, b = arr[i], arr[l]
        lo = jnp.minimum(a, b)
        hi = jnp.maximum(a, b)
        arr[i], arr[l] = (lo, hi) if asc else (hi, lo)
    return arr


def _select_body(xrx_ref, xry_ref, xrz_ref, cols_ref, out_ref):
    base = pl.program_id(0) * 1024
    rid = (jax.lax.broadcasted_iota(jnp.int32, (8, 128), 0) * 128
           + jax.lax.broadcasted_iota(jnp.int32, (8, 128), 1) + base)
    xx = xrx_ref[...]
    yy = xry_ref[...]
    zz = xrz_ref[...]
    r2 = RADIUS * RADIUS

    def chunk_body(c, state):
        cand = []
        for u in range(MAXK):
            j = c * MAXK + u
            dx = xx - cols_ref[0, j]
            dy = yy - cols_ref[1, j]
            dz = zz - cols_ref[2, j]
            d2 = dx * dx + dy * dy + dz * dz
            kb = jax.lax.bitcast_convert_type(d2, jnp.int32)
            key = jnp.bitwise_or(jnp.bitwise_and(kb, KEYHI), j)
            bad = (d2 > r2) | (rid == j)
            cand.append(jnp.where(bad, KEYBIG, key))
        cand = _apply_net(cand, _SORT32)
        merged = [jnp.minimum(state[i], cand[MAXK - 1 - i])
                  for i in range(MAXK)]
        return tuple(_apply_net(merged, _MERGE32))

    init = tuple(jnp.full((8, 128), KEYBIG, jnp.int32)
                 for _ in range(MAXK))
    state = jax.lax.fori_loop(0, NC // MAXK, chunk_body, init)
    for s in range(MAXK):
        out_ref[s] = state[s]


def _run_select(xrx, xry, xrz, cols):
    grid_spec = pl.GridSpec(
        grid=(NP // 1024,),
        in_specs=[
            pl.BlockSpec((8, 128), lambda i: (i, 0)),
            pl.BlockSpec((8, 128), lambda i: (i, 0)),
            pl.BlockSpec((8, 128), lambda i: (i, 0)),
            pl.BlockSpec(memory_space=pltpu.SMEM),
        ],
        out_specs=pl.BlockSpec((MAXK, 8, 128), lambda i: (0, i, 0)),
    )
    return pl.pallas_call(
        _select_body,
        grid_spec=grid_spec,
        out_shape=jax.ShapeDtypeStruct((MAXK, NP // 128, 128), jnp.int32),
    )(xrx, xry, xrz, cols)


def _gelu(x):
    return 0.5 * x * (1.0 + jax.lax.erf(x * 0.7071067811865476))


def _fused_body(feat_ref, g_ref, gf_ref,
                wei_ref, wej_ref, we1_ref, we2_ref,
                wg0_ref, g1w_ref,
                wu0a_ref, wu0b_ref, wu0c_ref, wu1_ref, wu2_ref,
                gl0_ref, gl1_ref, gl2_ref, glb_ref,
                s_ref, bias_ref, out_ref):
    be = BN * MAXK
    dot = lambda a, b: jax.lax.dot_general(
        a, b, (((1,), (0,)), ((), ())),
        precision=jax.lax.Precision.HIGHEST,
        preferred_element_type=jnp.float32)

    fx = feat_ref[...]
    feat = fx[:, :HID]
    xdu = fx[:, HID:HID + 1]
    g = g_ref[...]
    fj = g[:, :HID]
    d2c = g[:, HID:HID + 1]
    xju = g[:, HID + 1:HID + 2]
    vcol = g[:, HID + 2:HID + 3]

    bias = bias_ref[...]
    s = s_ref[...]
    b0 = bias[0:1]
    b1 = bias[1:2]
    b2 = bias[2:3]
    g0b = bias[3:4]
    u0b = bias[4:5]
    u1b = bias[5:6]
    u2b = bias[6:7, :HID]
    g1b = bias[7, 0]
    sres = bias[7, 1]
    w_d2 = s[0:1]
    w_a = s[1:2]
    w_b = s[2:3]

    nodepart = dot(feat, wei_ref[...]) + xdu * w_a
    npart_e = jnp.broadcast_to(
        nodepart[:, None, :], (BN, MAXK, MSG)).reshape(be, MSG)
    pre0 = npart_e + dot(fj, wej_ref[...]) + d2c * w_d2 + xju * w_b + b0
    h1 = _gelu(pre0)
    h2 = _gelu(dot(h1, we1_ref[...]) + b1)
    eh = dot(h2, we2_ref[...]) + b2

    gg = _gelu(dot(eh, wg0_ref[...]) + g0b)
    logit = jnp.sum(gg * g1w_ref[...], axis=1, keepdims=True) + g1b
    gate = jax.nn.sigmoid(logit)
    msg = eh * (gate * vcol)

    msum = jnp.sum(msg.reshape(BN, MAXK, MSG), axis=1)
    deg = jnp.sum(vcol.reshape(BN, MAXK, 1), axis=1)
    agg = msum / jnp.maximum(deg, 1.0)

    gf = gf_ref[...]
    c1 = _gelu(dot(gf, gl0_ref[...]) + glb_ref[0:1])
    c2 = _gelu(dot(c1, gl1_ref[...]) + glb_ref[1:2])
    gctx = dot(c2, gl2_ref[...]) + glb_ref[2:3]
    gterm = dot(gctx[0:1], wu0c_ref[...])

    upre = dot(feat, wu0a_ref[...]) + dot(agg, wu0b_ref[...]) + gterm + u0b
    v1 = _gelu(upre)
    v2 = _gelu(dot(v1, wu1_ref[...]) + u1b)
    dh = dot(v2, wu2_ref[...]) + u2b
    out_ref[...] = dh * sres


def _run_fused(featp, gp, gf, wei, wej, we1, we2, wg0, g1w,
               wu0a, wu0b, wu0c, wu1, wu2, gl0, gl1, gl2, glb, s, bias):
    nblk = NP // BN
    be = BN * MAXK
    full = lambda shp: pl.BlockSpec(shp, lambda i: tuple(0 for _ in shp))
    grid_spec = pl.GridSpec(
        grid=(nblk,),
        in_specs=[
            pl.BlockSpec((BN, 2 * HID), lambda i: (i, 0)),
            pl.BlockSpec((be, GW), lambda i: (i, 0)),
            full((8, HID)),
            full((HID, MSG)), full((HID, MSG)),
            full((MSG, MSG)), full((MSG, MSG)),
            full((MSG, MSG)), full((1, MSG)),
            full((HID, MSG)), full((MSG, MSG)), full((HID, MSG)),
            full((MSG, MSG)), full((MSG, HID)),
            full((HID, HID)), full((HID, HID)), full((HID, HID)),
            full((8, HID)),
            full((8, MSG)), full((8, MSG)),
        ],
        out_specs=pl.BlockSpec((BN, HID), lambda i: (i, 0)),
    )
    return pl.pallas_call(
        _fused_body,
        grid_spec=grid_spec,
        out_shape=jax.ShapeDtypeStruct((NP, HID), jnp.float32),
    )(featp, gp, gf, wei, wej, we1, we2, wg0, g1w,
      wu0a, wu0b, wu0c, wu1, wu2, gl0, gl1, gl2, glb, s, bias)


def kernel(coords, h, flow_dir, params):
    p = params
    f32 = jnp.float32
    x = coords[0]
    h0 = h[0]

    xpadr = jnp.pad(x, ((0, NP - N), (0, 0)), constant_values=10.0)
    xrx = xpadr[:, 0].reshape(NP // 128, 128)
    xry = xpadr[:, 1].reshape(NP // 128, 128)
    xrz = xpadr[:, 2].reshape(NP // 128, 128)
    cols = jnp.pad(x.T, ((0, 0), (0, NC - N)), constant_values=10.0)
    keys = _run_select(xrx, xry, xrz, cols)
    keys = keys.reshape(MAXK, NP).T[:N].reshape(-1)
    valid = keys < KEYBIG
    col = jnp.where(valid, jnp.bitwise_and(keys, 0x3FFF), 0)
    d2sel = jax.lax.bitcast_convert_type(
        jnp.bitwise_and(keys, KEYHI), jnp.float32)

    mean = jnp.mean(h0, axis=-1, keepdims=True)
    var = jnp.var(h0, axis=-1, keepdims=True)
    feat = (h0 - mean) / jnp.sqrt(var + 1e-05) * p['ln_g'] + p['ln_b']
    u = flow_dir[0]
    u = u / (jnp.linalg.norm(u) + 1e-08)
    xdu = x @ u

    fj = feat[col]
    xjuv = xdu[col]
    dist2 = jnp.where(valid, d2sel, 0.0)
    vf = valid.astype(f32)
    g = jnp.concatenate(
        [fj, dist2[:, None], xjuv[:, None], vf[:, None],
         jnp.zeros((N * MAXK, GW - HID - 3), f32)], axis=1)

    gp = jnp.pad(g, ((0, (NP - N) * MAXK), (0, 0)))
    fx = jnp.concatenate(
        [feat, xdu[:, None], jnp.zeros((N, HID - 1), f32)], axis=1)
    featp = jnp.pad(fx, ((0, NP - N), (0, 0)))
    gf = jnp.broadcast_to(jnp.mean(feat, axis=0, keepdims=True), (8, HID))

    e0w = p['e0_W']
    wei = e0w[0:HID] + e0w[2 * HID:3 * HID]
    wej = e0w[HID:2 * HID] - e0w[2 * HID:3 * HID]
    w_d2 = e0w[3 * HID + 0]
    w_a = e0w[3 * HID + 1] + e0w[3 * HID + 3]
    w_b = e0w[3 * HID + 2] - e0w[3 * HID + 3]
    s = jnp.zeros((8, MSG), f32).at[0].set(w_d2).at[1].set(w_a).at[2].set(w_b)
    bias = (jnp.zeros((8, MSG), f32)
            .at[0].set(p['e0_b']).at[1].set(p['e1_b']).at[2].set(p['e2_b'])
            .at[3].set(p['g0_b']).at[4].set(p['u0_b']).at[5].set(p['u1_b'])
            .at[6, :HID].set(p['u2_b'])
            .at[7, 0].set(p['g1_b'][0])
            .at[7, 1].set(jnp.tanh(p['res_scale'])))
    glb = (jnp.zeros((8, HID), f32)
           .at[0].set(p['gl0_b']).at[1].set(p['gl1_b']).at[2].set(p['gl2_b']))
    g1w = p['g1_W'].reshape(1, MSG)
    u0w = p['u0_W']
    wu0a = u0w[0:HID]
    wu0b = u0w[HID:HID + MSG]
    wu0c = u0w[HID + MSG:]

    out = _run_fused(featp, gp, gf, wei, wej, p['e1_W'], p['e2_W'],
                     p['g0_W'], g1w, wu0a, wu0b, wu0c, p['u1_W'], p['u2_W'],
                     p['gl0_W'], p['gl1_W'], p['gl2_W'], glb, s, bias)
    return out[:N][None]
